# baseline (speedup 1.0000x reference)
import functools

import jax
import jax.numpy as jnp
from jax.experimental import pallas as pl
from jax.experimental.pallas import tpu as pltpu
from jax.experimental.pallas import tpu_sc as plsc

B = 1024
L = 200
D = 64
H = 128
POOL = 8
DENSE_DIM = 256
HEAD_H = 256
MLP_H = 128
BB = 32
GW = 128

_SP_NAMES = ('user_tokens', 'context_tokens', 'candidate_post_tokens',
             'candidate_author_tokens', 'candidate_tokens')


def _sc_gather(token_table, tok_idx, sp_tables, sp_idx):
    nt = tok_idx.shape[1]
    ns = sp_idx[0].shape[1]
    mesh = plsc.VectorSubcoreMesh(core_axis_name="c", subcore_axis_name="s")
    out_type = ((jax.ShapeDtypeStruct((nt, D), jnp.float32),)
                + tuple(jax.ShapeDtypeStruct((ns, D), jnp.float32)
                        for _ in sp_tables))

    @functools.partial(
        pl.kernel, out_type=out_type, mesh=mesh,
        compiler_params=pltpu.CompilerParams(use_tc_tiling_on_sc=False))
    def k(tok_tab, tok_i, t0, t1, t2, t3, t4, i0, i1, i2, i3, i4,
          out_tok, o0, o1, o2, o3, o4):
        def tok_body(i_vmem, o_vmem):
            pltpu.sync_copy(tok_tab.at[i_vmem.at[0]], o_vmem)

        pltpu.emit_pipeline(
            tok_body,
            grid=(nt // GW,),
            in_specs=[pl.BlockSpec((1, GW), lambda i: (0, i))],
            out_specs=[pl.BlockSpec((GW, D), lambda i: (i, 0))],
            core_axis_name=("c", "s"),
            dimension_semantics=(pltpu.PARALLEL,),
        )(tok_i, out_tok)

        tabs = (t0, t1, t2, t3, t4)

        def sp_body(j0, j1, j2, j3, j4, p0, p1, p2, p3, p4):
            for tab, j, o in zip(tabs, (j0, j1, j2, j3, j4),
                                 (p0, p1, p2, p3, p4)):
                pltpu.sync_copy(tab.at[j.at[0]], o)

        pltpu.emit_pipeline(
            sp_body,
            grid=(ns // GW,),
            in_specs=[pl.BlockSpec((1, GW), lambda i: (0, i))] * 5,
            out_specs=[pl.BlockSpec((GW, D), lambda i: (i, 0))] * 5,
            core_axis_name=("c", "s"),
            dimension_semantics=(pltpu.PARALLEL,),
        )(i0, i1, i2, i3, i4, o0, o1, o2, o3, o4)

    return k(token_table, tok_idx, *sp_tables, *sp_idx)


def _dense_kernel(ecat_ref, len_ref, pu_ref, pc_ref, pcp_ref, pca_ref,
                  pct_ref, dn_ref,
                  whist_ref, lnh_ref, fuser_ref, lnu_ref, fcand_ref, lnc_ref,
                  wdense_ref, lnd_ref, w1ac_ref, w1h_ref, w1p_ref, attp_ref,
                  hw1_ref, hb1_ref, hw2_ref, hb2_ref, hw3_ref, scal_ref,
                  out_ref):
    f32 = jnp.float32

    def dot(a, b):
        return jax.lax.dot_general(a, b, (((1,), (0,)), ((), ())),
                                   preferred_element_type=f32)

    def ln_act(y, lnref):
        y = y + lnref[0:1, :]
        m = jnp.mean(y, axis=-1, keepdims=True)
        v = jnp.mean((y - m) ** 2, axis=-1, keepdims=True)
        y = (y - m) * jax.lax.rsqrt(v + 1e-5) * lnref[1:2, :] + lnref[2:3, :]
        return jax.nn.gelu(y)

    hist2 = ln_act(dot(ecat_ref[...], whist_ref[...]), lnh_ref)
    lens = len_ref[...]
    iota3 = jax.lax.broadcasted_iota(jnp.int32, (BB, L, 1), 1)
    mask3 = (iota3 < lens[:, :, None]).astype(f32)
    h3 = hist2.reshape(BB, L, H) * mask3
    hist2m = h3.reshape(BB * L, H)

    p_u = jnp.sum(pu_ref[...], axis=1)
    p_c = jnp.sum(pc_ref[...], axis=1)
    p_cp = jnp.sum(pcp_ref[...], axis=1)
    p_ca = jnp.sum(pca_ref[...], axis=1)
    p_ct = jnp.sum(pct_ref[...], axis=1)

    fuser = fuser_ref[...]
    user = ln_act(dot(p_u, fuser[0:64]) + dot(p_c, fuser[64:128]), lnu_ref)
    fcand = fcand_ref[...]
    cand = ln_act(dot(p_cp, fcand[0:64]) + dot(p_ca, fcand[64:128])
                  + dot(p_ct, fcand[128:192]), lnc_ref)
    dense = ln_act(dot(dn_ref[...], wdense_ref[...]), lnd_ref)

    att_a = scal_ref[0:1, 0:1]
    att_b2 = scal_ref[0:1, 1:2]
    a_row = dot(cand, w1ac_ref[...])
    hterm = dot(hist2m, w1h_ref[...])
    prod = h3 * cand[:, None, :]
    pterm = dot(prod.reshape(BB * L, H), w1p_ref[...])
    pre3 = ((hterm + pterm).reshape(BB, L, H) + a_row[:, None, :]
            + attp_ref[0:1, :][None])
    sact = jnp.where(pre3 >= 0, pre3, pre3 * att_a[:, :, None])
    s = jnp.sum(sact * attp_ref[1:2, :][None], axis=-1) + att_b2

    iota2 = jax.lax.broadcasted_iota(jnp.int32, (BB, L), 1)
    mask2 = iota2 < lens
    s = jnp.where(mask2, s, -1e9)
    smax = jnp.max(s, axis=-1, keepdims=True)
    e = jnp.exp(s - smax)
    w = e / jnp.sum(e, axis=-1, keepdims=True)
    context = jnp.sum(w[:, :, None] * h3, axis=1)
    denom = jnp.maximum(jnp.sum(mask2.astype(f32), axis=-1, keepdims=True),
                        1.0)
    summary = jnp.sum(h3, axis=1) / denom

    a1 = scal_ref[0:1, 2:3]
    a2 = scal_ref[0:1, 3:4]
    b3 = scal_ref[0:1, 4:5]
    hw1 = hw1_ref[...]
    pieces = (cand, context, summary, user, dense, cand * context,
              jnp.abs(cand - user))
    acc = dot(pieces[0], hw1[0:H])
    for kk in range(1, 7):
        acc = acc + dot(pieces[kk], hw1[kk * H:(kk + 1) * H])
    h1 = acc + hb1_ref[...]
    h1 = jnp.where(h1 >= 0, h1, h1 * a1)
    h2 = dot(h1, hw2_ref[...]) + hb2_ref[...]
    h2 = jnp.where(h2 >= 0, h2, h2 * a2)
    out_ref[...] = jnp.sum(h2 * hw3_ref[...], axis=-1, keepdims=True) + b3


def _fold_weights(p):
    tok_w = p['tok_W']
    tok_b = p['tok_b']

    def pack_ln(bias, g, beta):
        z = jnp.zeros((8, H), jnp.float32)
        return z.at[0].set(bias).at[1].set(g).at[2].set(beta)

    def fold(w_big, n):
        blocks = [tok_w @ w_big[k * H:(k + 1) * H] for k in range(n)]
        bias = sum(tok_b @ w_big[k * H:(k + 1) * H] for k in range(n))
        return jnp.concatenate(blocks, axis=0), bias

    whist, bh = fold(p['hist_W'], 3)
    fuser, bu = fold(p['user_W'], 2)
    fcand, bc = fold(p['cand_W'], 3)

    w1 = p['att_W1']
    w1ac = w1[0:H] + w1[2 * H:3 * H]
    w1h = w1[H:2 * H] - w1[2 * H:3 * H]
    w1p = w1[3 * H:4 * H]
    attp = jnp.zeros((8, H), jnp.float32)
    attp = attp.at[0].set(p['att_b1']).at[1].set(p['att_W2'].reshape(H))
    scal = jnp.concatenate([
        jnp.stack([p['att_a'], p['att_b2'][0], p['head_a1'], p['head_a2'],
                   p['head_b3'][0]]),
        jnp.zeros((3,), jnp.float32)]).reshape(1, 8)

    return dict(
        whist=whist,
        lnh=pack_ln(p['hist_b'] + bh, p['hist_g'], p['hist_beta']),
        fuser=fuser,
        lnu=pack_ln(p['user_b'] + bu, p['user_g'], p['user_beta']),
        fcand=fcand,
        lnc=pack_ln(p['cand_b'] + bc, p['cand_g'], p['cand_beta']),
        wdense=p['dense_W'],
        lnd=pack_ln(p['dense_b'], p['dense_g'], p['dense_beta']),
        w1ac=w1ac, w1h=w1h, w1p=w1p, attp=attp,
        hw1=p['head_W1'], hb1=p['head_b1'].reshape(1, HEAD_H),
        hw2=p['head_W2'], hb2=p['head_b2'].reshape(1, MLP_H),
        hw3=p['head_W3'].reshape(1, MLP_H),
        scal=scal,
    )


def _dense_forward(ecat, lengths, pooled, dense_features, p):
    fw = _fold_weights(p)
    lens2 = lengths.astype(jnp.int32).reshape(B, 1)

    def full(shp):
        return pl.BlockSpec(shp, lambda i: tuple(0 for _ in shp))

    out = pl.pallas_call(
        _dense_kernel,
        grid=(B // BB,),
        in_specs=[
            pl.BlockSpec((BB * L, 3 * D), lambda i: (i, 0)),
            pl.BlockSpec((BB, 1), lambda i: (i, 0)),
            pl.BlockSpec((BB, POOL, D), lambda i: (i, 0, 0)),
            pl.BlockSpec((BB, POOL, D), lambda i: (i, 0, 0)),
            pl.BlockSpec((BB, POOL, D), lambda i: (i, 0, 0)),
            pl.BlockSpec((BB, POOL, D), lambda i: (i, 0, 0)),
            pl.BlockSpec((BB, POOL, D), lambda i: (i, 0, 0)),
            pl.BlockSpec((BB, DENSE_DIM), lambda i: (i, 0)),
            full((3 * D, H)), full((8, H)), full((2 * D, H)), full((8, H)),
            full((3 * D, H)), full((8, H)), full((DENSE_DIM, H)), full((8, H)),
            full((H, H)), full((H, H)), full((H, H)), full((8, H)),
            full((7 * H, HEAD_H)), full((1, HEAD_H)), full((HEAD_H, MLP_H)),
            full((1, MLP_H)), full((1, MLP_H)), full((1, 8)),
        ],
        out_specs=pl.BlockSpec((BB, 1), lambda i: (i, 0)),
        out_shape=jax.ShapeDtypeStruct((B, 1), jnp.float32),
    )(ecat, lens2, *pooled, dense_features,
      fw['whist'], fw['lnh'], fw['fuser'], fw['lnu'], fw['fcand'], fw['lnc'],
      fw['wdense'], fw['lnd'], fw['w1ac'], fw['w1h'], fw['w1p'], fw['attp'],
      fw['hw1'], fw['hb1'], fw['hw2'], fw['hb2'], fw['hw3'], fw['scal'])
    return out.reshape(B)


def kernel(history_post_tokens, history_author_tokens, history_action_tokens,
           history_lengths, user_tokens_idx, context_tokens_idx,
           candidate_tokens_idx, candidate_post_tokens_idx,
           candidate_author_tokens_idx, dense_features, params):
    p = params
    tok_idx = jnp.stack([history_post_tokens, history_author_tokens,
                         history_action_tokens], axis=-1)
    tok_idx = tok_idx.astype(jnp.int32).reshape(1, B * L * 3)
    sp_idx_map = {
        'user_tokens': user_tokens_idx,
        'context_tokens': context_tokens_idx,
        'candidate_post_tokens': candidate_post_tokens_idx,
        'candidate_author_tokens': candidate_author_tokens_idx,
        'candidate_tokens': candidate_tokens_idx,
    }
    sp_tables = [p[n + '_table'] for n in _SP_NAMES]
    sp_idx = [sp_idx_map[n].astype(jnp.int32).reshape(1, B * POOL)
              for n in _SP_NAMES]

    gathered = _sc_gather(p['token_table'], tok_idx, sp_tables, sp_idx)
    ecat = gathered[0].reshape(B * L, 3 * D)
    pooled = [g.reshape(B, POOL, D) for g in gathered[1:]]
    return _dense_forward(ecat, history_lengths, pooled, dense_features, p)
